# Initial kernel scaffold; baseline (speedup 1.0000x reference)
import jax
import jax.numpy as jnp
from jax.experimental import pallas as pl


def kernel(colemb, tabemb, graph_tensor, We, be, Wi, Wh, bi, bh):
    raise NotImplementedError("write your pallas kernel here")


# speedup vs baseline: 22.0719x; 22.0719x over previous
import functools

import jax
import jax.numpy as jnp
from jax import lax
from jax.experimental import pallas as pl
from jax.experimental.pallas import tpu as pltpu

_B, _NT, _NCOL, _D = 256, 32, 96, 128
_ET, _E, _T = 4, 512, 3
_N = _NT + _NCOL


def _adj_kernel(src_ref, dst_ref, adj_ref):
    src = src_ref[0]
    dst = dst_ref[0]
    ids_en = lax.broadcasted_iota(jnp.int32, (_E, _N), 1)
    ids_ne = lax.broadcasted_iota(jnp.int32, (_N, _E), 0)
    for e in range(_ET):
        src_oh = (src[e][:, None] == ids_en).astype(jnp.float32)
        dst_oh_t = (dst[e][None, :] == ids_ne).astype(jnp.float32)
        blk = jnp.dot(dst_oh_t, src_oh, preferred_element_type=jnp.float32)
        adj_ref[0, :, e * _N:(e + 1) * _N] = blk


def _gnn_kernel(h0_ref, adj_ref, We_ref, be_ref, Wi_ref, Wh_ref, bi_ref,
                bh_ref, out_ref, G):
    h = h0_ref[...].reshape(G * _N, _D)
    We = We_ref[...]
    be = be_ref[...]
    Wi = Wi_ref[...]
    Wh = Wh_ref[...]
    bi = bi_ref[...]
    bh = bh_ref[...]
    for _ in range(_T):
        ms = [jnp.dot(h, We[e], preferred_element_type=jnp.float32)
              + be[e][None, :] for e in range(_ET)]
        a_rows = []
        for g in range(G):
            mcat = jnp.concatenate(
                [ms[e][g * _N:(g + 1) * _N, :] for e in range(_ET)], axis=0)
            a_rows.append(jnp.dot(adj_ref[g], mcat,
                                  preferred_element_type=jnp.float32))
        a = jnp.concatenate(a_rows, axis=0)
        gi = jnp.dot(a, Wi, preferred_element_type=jnp.float32) + bi
        gh = jnp.dot(h, Wh, preferred_element_type=jnp.float32) + bh
        r = jax.nn.sigmoid(gi[:, :_D] + gh[:, :_D])
        z = jax.nn.sigmoid(gi[:, _D:2 * _D] + gh[:, _D:2 * _D])
        n = jnp.tanh(gi[:, 2 * _D:] + r * gh[:, 2 * _D:])
        h = (1.0 - z) * n + z * h
    out_ref[...] = h.reshape(G, _N, _D)


def _build_adj(src, dst, interpret=False):
    return pl.pallas_call(
        _adj_kernel,
        grid=(_B,),
        in_specs=[
            pl.BlockSpec((1, _ET, _E), lambda i: (i, 0, 0)),
            pl.BlockSpec((1, _ET, _E), lambda i: (i, 0, 0)),
        ],
        out_specs=pl.BlockSpec((1, _N, _ET * _N), lambda i: (i, 0, 0)),
        out_shape=jax.ShapeDtypeStruct((_B, _N, _ET * _N), jnp.float32),
        interpret=interpret,
    )(src, dst)


def _run_gnn(h0, adj, We, be, Wi, Wh, bi, bh, G=8, interpret=False):
    nblk = _B // G
    return pl.pallas_call(
        functools.partial(_gnn_kernel, G=G),
        grid=(nblk,),
        in_specs=[
            pl.BlockSpec((G, _N, _D), lambda i: (i, 0, 0)),
            pl.BlockSpec((G, _N, _ET * _N), lambda i: (i, 0, 0)),
            pl.BlockSpec((_ET, _D, _D), lambda i: (0, 0, 0)),
            pl.BlockSpec((_ET, _D), lambda i: (0, 0)),
            pl.BlockSpec((_D, 3 * _D), lambda i: (0, 0)),
            pl.BlockSpec((_D, 3 * _D), lambda i: (0, 0)),
            pl.BlockSpec((1, 3 * _D), lambda i: (0, 0)),
            pl.BlockSpec((1, 3 * _D), lambda i: (0, 0)),
        ],
        out_specs=pl.BlockSpec((G, _N, _D), lambda i: (i, 0, 0)),
        out_shape=jax.ShapeDtypeStruct((_B, _N, _D), jnp.float32),
        interpret=interpret,
    )(h0, adj, We, be, Wi, Wh, bi, bh)


def kernel(colemb, tabemb, graph_tensor, We, be, Wi, Wh, bi, bh,
           interpret=False):
    h0 = jnp.concatenate([tabemb, colemb], axis=1)
    src = graph_tensor[..., 0]
    dst = graph_tensor[..., 1]
    adj = _build_adj(src, dst, interpret=interpret)
    out = _run_gnn(h0, adj, We, be, Wi, Wh,
                   bi.reshape(1, -1), bh.reshape(1, -1),
                   interpret=interpret)
    return out[:, _NT:, :], out[:, :_NT, :]


# speedup vs baseline: 24.7509x; 1.1214x over previous
import functools

import jax
import jax.numpy as jnp
from jax import lax
from jax.experimental import pallas as pl
from jax.experimental.pallas import tpu as pltpu
from jax.experimental.pallas import tpu_sc as plsc

_B, _NT, _NCOL, _D = 256, 32, 96, 128
_ET, _E, _T = 4, 512, 3
_N = _NT + _NCOL
_ROW = _ET * _N
_AW = _N * _ROW
_NC, _NS = 2, 16
_GPW = _B // (_NC * _NS)
_ZW = 4096


def _adj_sc_body(src_hbm, dst_hbm, adj_hbm, src_v, dst_v, idx_v, ones_v,
                 zeros_v, acc_shared):
    cid = lax.axis_index("c")
    sid = lax.axis_index("s")
    wid = sid * _NC + cid
    srow = sid * _AW

    def fill_body(k, _):
        zeros_v[pl.ds(k * 16, 16)] = jnp.zeros((16,), jnp.float32)
        return 0

    lax.fori_loop(0, _ZW // 16, fill_body, 0)
    for i in range(8):
        ones_v[pl.ds(i * 16, 16)] = jnp.full((16,), 1.0, jnp.float32)

    def per_graph(j, _):
        g = wid * _GPW + j
        pltpu.sync_copy(src_hbm.at[g], src_v)
        pltpu.sync_copy(dst_hbm.at[g], dst_v)
        for z in range(_AW // _ZW):
            pltpu.sync_copy(zeros_v, acc_shared.at[pl.ds(srow + z * _ZW, _ZW)])

        def idx_body(c, _):
            s16 = src_v[pl.ds(c * 16, 16)]
            d16 = dst_v[pl.ds(c * 16, 16)]
            e = c // (_E // 16)
            v = d16 * _ROW + e * _N + s16 + srow
            idx_v[c // 8, pl.ds((c % 8) * 16, 16)] = v
            return 0

        lax.fori_loop(0, (_ET * _E) // 16, idx_body, 0)
        for r in range(16):
            pltpu.sync_copy(ones_v, acc_shared.at[idx_v.at[r]], add=True)
        pltpu.sync_copy(acc_shared.at[pl.ds(srow, _AW)], adj_hbm.at[g])
        return 0

    lax.fori_loop(0, _GPW, per_graph, 0)


def _build_adj_sc(src, dst):
    mesh = plsc.VectorSubcoreMesh(core_axis_name="c", subcore_axis_name="s",
                                  num_cores=_NC, num_subcores=_NS)
    kfn = pl.kernel(
        _adj_sc_body,
        out_type=jax.ShapeDtypeStruct((_B, _AW), jnp.float32),
        mesh=mesh,
        scratch_types=[
            pltpu.VMEM((_ET * _E,), jnp.int32),
            pltpu.VMEM((_ET * _E,), jnp.int32),
            pltpu.VMEM((16, 128), jnp.int32),
            pltpu.VMEM((128,), jnp.float32),
            pltpu.VMEM((_ZW,), jnp.float32),
            pltpu.VMEM_SHARED((_NS * _AW,), jnp.float32),
        ],
    )
    return kfn(src, dst).reshape(_B, _N, _ROW)


def _gnn_kernel(h0_ref, adj_ref, We_ref, be_ref, Wi_ref, Wh_ref, bi_ref,
                bh_ref, out_ref, G):
    h = h0_ref[...].reshape(G * _N, _D)
    We = We_ref[...]
    be = be_ref[...]
    Wi = Wi_ref[...]
    Wh = Wh_ref[...]
    bi = bi_ref[...]
    bh = bh_ref[...]
    for _ in range(_T):
        ms = [jnp.dot(h, We[e], preferred_element_type=jnp.float32)
              + be[e][None, :] for e in range(_ET)]
        a_rows = []
        for g in range(G):
            mcat = jnp.concatenate(
                [ms[e][g * _N:(g + 1) * _N, :] for e in range(_ET)], axis=0)
            a_rows.append(jnp.dot(adj_ref[g], mcat,
                                  preferred_element_type=jnp.float32))
        a = jnp.concatenate(a_rows, axis=0)
        gi = jnp.dot(a, Wi, preferred_element_type=jnp.float32) + bi
        gh = jnp.dot(h, Wh, preferred_element_type=jnp.float32) + bh
        r = jax.nn.sigmoid(gi[:, :_D] + gh[:, :_D])
        z = jax.nn.sigmoid(gi[:, _D:2 * _D] + gh[:, _D:2 * _D])
        n = jnp.tanh(gi[:, 2 * _D:] + r * gh[:, 2 * _D:])
        h = (1.0 - z) * n + z * h
    out_ref[...] = h.reshape(G, _N, _D)


def _run_gnn(h0, adj, We, be, Wi, Wh, bi, bh, G=8):
    nblk = _B // G
    return pl.pallas_call(
        functools.partial(_gnn_kernel, G=G),
        grid=(nblk,),
        in_specs=[
            pl.BlockSpec((G, _N, _D), lambda i: (i, 0, 0)),
            pl.BlockSpec((G, _N, _ROW), lambda i: (i, 0, 0)),
            pl.BlockSpec((_ET, _D, _D), lambda i: (0, 0, 0)),
            pl.BlockSpec((_ET, _D), lambda i: (0, 0)),
            pl.BlockSpec((_D, 3 * _D), lambda i: (0, 0)),
            pl.BlockSpec((_D, 3 * _D), lambda i: (0, 0)),
            pl.BlockSpec((1, 3 * _D), lambda i: (0, 0)),
            pl.BlockSpec((1, 3 * _D), lambda i: (0, 0)),
        ],
        out_specs=pl.BlockSpec((G, _N, _D), lambda i: (i, 0, 0)),
        out_shape=jax.ShapeDtypeStruct((_B, _N, _D), jnp.float32),
    )(h0, adj, We, be, Wi, Wh, bi, bh)


def kernel(colemb, tabemb, graph_tensor, We, be, Wi, Wh, bi, bh):
    h0 = jnp.concatenate([tabemb, colemb], axis=1)
    src = graph_tensor[..., 0].reshape(_B, _ET * _E)
    dst = graph_tensor[..., 1].reshape(_B, _ET * _E)
    adj = _build_adj_sc(src, dst)
    out = _run_gnn(h0, adj, We, be, Wi, Wh,
                   bi.reshape(1, -1), bh.reshape(1, -1))
    return out[:, _NT:, :], out[:, :_NT, :]


# speedup vs baseline: 25.3060x; 1.0224x over previous
import functools

import jax
import jax.numpy as jnp
from jax import lax
from jax.experimental import pallas as pl
from jax.experimental.pallas import tpu as pltpu
from jax.experimental.pallas import tpu_sc as plsc

_B, _NT, _NCOL, _D = 256, 32, 96, 128
_ET, _E, _T = 4, 512, 3
_N = _NT + _NCOL
_ROW = _ET * _N
_AW = _N * _ROW
_NC, _NS = 2, 16
_GPW = _B // (_NC * _NS)
_ZW = 4096


def _adj_sc_body(src_hbm, dst_hbm, adj_hbm, src_v, dst_v, idx_v, ones_v,
                 zeros_v, acc_shared):
    cid = lax.axis_index("c")
    sid = lax.axis_index("s")
    wid = sid * _NC + cid
    srow = sid * _AW

    def fill_body(k, _):
        zeros_v[pl.ds(k * 16, 16)] = jnp.zeros((16,), jnp.float32)
        return 0

    lax.fori_loop(0, _ZW // 16, fill_body, 0)
    for i in range(8):
        ones_v[pl.ds(i * 16, 16)] = jnp.full((16,), 1.0, jnp.float32)

    def per_graph(j, _):
        g = wid * _GPW + j
        pltpu.sync_copy(src_hbm.at[g], src_v)
        pltpu.sync_copy(dst_hbm.at[g], dst_v)
        for z in range(_AW // _ZW):
            pltpu.sync_copy(zeros_v, acc_shared.at[pl.ds(srow + z * _ZW, _ZW)])

        def idx_body(c, _):
            s16 = src_v[pl.ds(c * 16, 16)]
            d16 = dst_v[pl.ds(c * 16, 16)]
            e = c // (_E // 16)
            v = d16 * _ROW + e * _N + s16 + srow
            idx_v[c // 8, pl.ds((c % 8) * 16, 16)] = v
            return 0

        lax.fori_loop(0, (_ET * _E) // 16, idx_body, 0)
        for r in range(16):
            pltpu.sync_copy(ones_v, acc_shared.at[idx_v.at[r]], add=True)
        pltpu.sync_copy(acc_shared.at[pl.ds(srow, _AW)], adj_hbm.at[g])
        return 0

    lax.fori_loop(0, _GPW, per_graph, 0)


def _build_adj_sc(src, dst):
    mesh = plsc.VectorSubcoreMesh(core_axis_name="c", subcore_axis_name="s",
                                  num_cores=_NC, num_subcores=_NS)
    kfn = pl.kernel(
        _adj_sc_body,
        out_type=jax.ShapeDtypeStruct((_B, _AW), jnp.float32),
        mesh=mesh,
        scratch_types=[
            pltpu.VMEM((_ET * _E,), jnp.int32),
            pltpu.VMEM((_ET * _E,), jnp.int32),
            pltpu.VMEM((16, 128), jnp.int32),
            pltpu.VMEM((128,), jnp.float32),
            pltpu.VMEM((_ZW,), jnp.float32),
            pltpu.VMEM_SHARED((_NS * _AW,), jnp.float32),
        ],
    )
    return kfn(src, dst).reshape(_B, _N, _ROW)


def _gnn_kernel(tab_ref, col_ref, adj_ref, We_ref, be_ref, Wi_ref, Wh_ref,
                bi_ref, bh_ref, out_ref, G):
    bf = jnp.bfloat16
    h = jnp.concatenate([tab_ref[...], col_ref[...]], axis=1)
    h = h.reshape(G * _N, _D)
    We = We_ref[...].astype(bf)
    be = be_ref[...]
    Wi = Wi_ref[...].astype(bf)
    Wh = Wh_ref[...].astype(bf)
    bi = bi_ref[...]
    bh = bh_ref[...]
    adj = adj_ref[...].astype(bf)
    for _ in range(_T):
        hb = h.astype(bf)
        ms = [jnp.dot(hb, We[e], preferred_element_type=jnp.float32)
              + be[e][None, :] for e in range(_ET)]
        a_rows = []
        for g in range(G):
            mcat = jnp.concatenate(
                [ms[e][g * _N:(g + 1) * _N, :] for e in range(_ET)],
                axis=0).astype(bf)
            a_rows.append(jnp.dot(adj[g], mcat,
                                  preferred_element_type=jnp.float32))
        a = jnp.concatenate(a_rows, axis=0)
        gi = jnp.dot(a.astype(bf), Wi, preferred_element_type=jnp.float32) + bi
        gh = jnp.dot(hb, Wh, preferred_element_type=jnp.float32) + bh
        r = jax.nn.sigmoid(gi[:, :_D] + gh[:, :_D])
        z = jax.nn.sigmoid(gi[:, _D:2 * _D] + gh[:, _D:2 * _D])
        n = jnp.tanh(gi[:, 2 * _D:] + r * gh[:, 2 * _D:])
        h = (1.0 - z) * n + z * h
    out_ref[...] = h.reshape(G, _N, _D)


def _run_gnn(tabemb, colemb, adj, We, be, Wi, Wh, bi, bh, G=8):
    nblk = _B // G
    return pl.pallas_call(
        functools.partial(_gnn_kernel, G=G),
        grid=(nblk,),
        in_specs=[
            pl.BlockSpec((G, _NT, _D), lambda i: (i, 0, 0)),
            pl.BlockSpec((G, _NCOL, _D), lambda i: (i, 0, 0)),
            pl.BlockSpec((G, _N, _ROW), lambda i: (i, 0, 0)),
            pl.BlockSpec((_ET, _D, _D), lambda i: (0, 0, 0)),
            pl.BlockSpec((_ET, _D), lambda i: (0, 0)),
            pl.BlockSpec((_D, 3 * _D), lambda i: (0, 0)),
            pl.BlockSpec((_D, 3 * _D), lambda i: (0, 0)),
            pl.BlockSpec((1, 3 * _D), lambda i: (0, 0)),
            pl.BlockSpec((1, 3 * _D), lambda i: (0, 0)),
        ],
        out_specs=pl.BlockSpec((G, _N, _D), lambda i: (i, 0, 0)),
        out_shape=jax.ShapeDtypeStruct((_B, _N, _D), jnp.float32),
    )(tabemb, colemb, adj, We, be, Wi, Wh, bi, bh)


def kernel(colemb, tabemb, graph_tensor, We, be, Wi, Wh, bi, bh):
    src = graph_tensor[..., 0].reshape(_B, _ET * _E)
    dst = graph_tensor[..., 1].reshape(_B, _ET * _E)
    adj = _build_adj_sc(src, dst)
    out = _run_gnn(tabemb, colemb, adj, We, be, Wi, Wh,
                   bi.reshape(1, -1), bh.reshape(1, -1))
    return out[:, _NT:, :], out[:, :_NT, :]


# speedup vs baseline: 26.9679x; 1.0657x over previous
import functools

import jax
import jax.numpy as jnp
from jax import lax
from jax.experimental import pallas as pl
from jax.experimental.pallas import tpu as pltpu
from jax.experimental.pallas import tpu_sc as plsc

_B, _NT, _NCOL, _D = 256, 32, 96, 128
_ET, _E, _T = 4, 512, 3
_N = _NT + _NCOL
_ROW = _ET * _N
_AW = _N * _ROW
_NC, _NS = 2, 16
_GPW = _B // (_NC * _NS)
_ZW = 16384


def _adj_sc_body(sd_hbm, adj_hbm, sd_v, idx_v, ones_v, zeros_v, acc_shared):
    cid = lax.axis_index("c")
    sid = lax.axis_index("s")
    wid = sid * _NC + cid
    srow = sid * _AW
    ne = _ET * _E

    def fill_body(k, _):
        zeros_v[pl.ds(k * 16, 16)] = jnp.zeros((16,), jnp.float32)
        return 0

    lax.fori_loop(0, _ZW // 16, fill_body, 0)
    for i in range(8):
        ones_v[pl.ds(i * 16, 16)] = jnp.full((16,), 1.0, jnp.float32)

    def per_graph(j, _):
        g = wid * _GPW + j
        pltpu.sync_copy(sd_hbm.at[g], sd_v)

        def idx_body(c, _):
            s16 = sd_v[pl.ds(c * 16, 16)]
            d16 = sd_v[pl.ds(ne + c * 16, 16)]
            e = c // (_E // 16)
            v = d16 * _ROW + e * _N + s16 + srow
            idx_v[c // 8, pl.ds((c % 8) * 16, 16)] = v
            return 0

        lax.fori_loop(0, ne // 16, idx_body, 0)
        for z in range(_AW // _ZW):
            pltpu.sync_copy(zeros_v, acc_shared.at[pl.ds(srow + z * _ZW, _ZW)])
        for r in range(16):
            pltpu.sync_copy(ones_v, acc_shared.at[idx_v.at[r]], add=True)
        pltpu.sync_copy(acc_shared.at[pl.ds(srow, _AW)], adj_hbm.at[g])
        return 0

    lax.fori_loop(0, _GPW, per_graph, 0)


def _build_adj_sc(sd):
    mesh = plsc.VectorSubcoreMesh(core_axis_name="c", subcore_axis_name="s",
                                  num_cores=_NC, num_subcores=_NS)
    kfn = pl.kernel(
        _adj_sc_body,
        out_type=jax.ShapeDtypeStruct((_B, _AW), jnp.float32),
        mesh=mesh,
        scratch_types=[
            pltpu.VMEM((2 * _ET * _E,), jnp.int32),
            pltpu.VMEM((16, 128), jnp.int32),
            pltpu.VMEM((128,), jnp.float32),
            pltpu.VMEM((_ZW,), jnp.float32),
            pltpu.VMEM_SHARED((_NS * _AW,), jnp.float32),
        ],
    )
    return kfn(sd).reshape(_B, _N, _ROW)


def _gnn_kernel(tab_ref, col_ref, adj_ref, We_ref, be_ref, Wi_ref, Wh_ref,
                bi_ref, bh_ref, out_ref, G):
    bf = jnp.bfloat16
    h = jnp.concatenate([tab_ref[...], col_ref[...]], axis=1)
    h = h.reshape(G * _N, _D)
    We = We_ref[...].astype(bf)
    be = be_ref[...]
    Wi = Wi_ref[...].astype(bf)
    Wh = Wh_ref[...].astype(bf)
    bi = bi_ref[...]
    bh = bh_ref[...]
    adj = adj_ref[...].astype(bf)
    for _ in range(_T):
        hb = h.astype(bf)
        ms = [jnp.dot(hb, We[e], preferred_element_type=jnp.float32)
              + be[e][None, :] for e in range(_ET)]
        a_rows = []
        for g in range(G):
            mcat = jnp.concatenate(
                [ms[e][g * _N:(g + 1) * _N, :] for e in range(_ET)],
                axis=0).astype(bf)
            a_rows.append(jnp.dot(adj[g], mcat,
                                  preferred_element_type=jnp.float32))
        a = jnp.concatenate(a_rows, axis=0)
        gi = jnp.dot(a.astype(bf), Wi, preferred_element_type=jnp.float32) + bi
        gh = jnp.dot(hb, Wh, preferred_element_type=jnp.float32) + bh
        r = jax.nn.sigmoid(gi[:, :_D] + gh[:, :_D])
        z = jax.nn.sigmoid(gi[:, _D:2 * _D] + gh[:, _D:2 * _D])
        n = jnp.tanh(gi[:, 2 * _D:] + r * gh[:, 2 * _D:])
        h = (1.0 - z) * n + z * h
    out_ref[...] = h.reshape(G, _N, _D)


def _run_gnn(tabemb, colemb, adj, We, be, Wi, Wh, bi, bh, G=8):
    nblk = _B // G
    return pl.pallas_call(
        functools.partial(_gnn_kernel, G=G),
        grid=(nblk,),
        in_specs=[
            pl.BlockSpec((G, _NT, _D), lambda i: (i, 0, 0)),
            pl.BlockSpec((G, _NCOL, _D), lambda i: (i, 0, 0)),
            pl.BlockSpec((G, _N, _ROW), lambda i: (i, 0, 0)),
            pl.BlockSpec((_ET, _D, _D), lambda i: (0, 0, 0)),
            pl.BlockSpec((_ET, _D), lambda i: (0, 0)),
            pl.BlockSpec((_D, 3 * _D), lambda i: (0, 0)),
            pl.BlockSpec((_D, 3 * _D), lambda i: (0, 0)),
            pl.BlockSpec((1, 3 * _D), lambda i: (0, 0)),
            pl.BlockSpec((1, 3 * _D), lambda i: (0, 0)),
        ],
        out_specs=pl.BlockSpec((G, _N, _D), lambda i: (i, 0, 0)),
        out_shape=jax.ShapeDtypeStruct((_B, _N, _D), jnp.float32),
    )(tabemb, colemb, adj, We, be, Wi, Wh, bi, bh)


def kernel(colemb, tabemb, graph_tensor, We, be, Wi, Wh, bi, bh):
    sd = jnp.moveaxis(graph_tensor.reshape(_B, _ET * _E, 2), 2, 1)
    adj = _build_adj_sc(sd.reshape(_B, 2 * _ET * _E))
    out = _run_gnn(tabemb, colemb, adj, We, be, Wi, Wh,
                   bi.reshape(1, -1), bh.reshape(1, -1))
    return out[:, _NT:, :], out[:, :_NT, :]


# speedup vs baseline: 27.7506x; 1.0290x over previous
import functools

import jax
import jax.numpy as jnp
from jax import lax
from jax.experimental import pallas as pl
from jax.experimental.pallas import tpu as pltpu
from jax.experimental.pallas import tpu_sc as plsc

_B, _NT, _NCOL, _D = 256, 32, 96, 128
_ET, _E, _T = 4, 512, 3
_N = _NT + _NCOL
_ROW = _ET * _N
_AW = _N * _ROW
_NC, _NS = 2, 16
_GPW = _B // (_NC * _NS)
_ZW = 16384


def _adj_sc_body(sd_hbm, adj_hbm, sd_v, idx_v, ones_v, zeros_v, acc_shared):
    cid = lax.axis_index("c")
    sid = lax.axis_index("s")
    wid = sid * _NC + cid
    srow = sid * _AW
    ne = _ET * _E

    def fill_body(k, _):
        zeros_v[pl.ds(k * 16, 16)] = jnp.zeros((16,), jnp.float32)
        return 0

    lax.fori_loop(0, _ZW // 16, fill_body, 0)

    def ones_body(k, _):
        ones_v[pl.ds(k * 16, 16)] = jnp.full((16,), 1.0, jnp.float32)
        return 0

    lax.fori_loop(0, ne // 16, ones_body, 0)

    def per_graph(j, _):
        g = wid * _GPW + j
        pltpu.sync_copy(sd_hbm.at[g], sd_v)

        def idx_body(c, _):
            s16 = sd_v[pl.ds(c * 16, 16)]
            d16 = sd_v[pl.ds(ne + c * 16, 16)]
            e = c // (_E // 16)
            v = d16 * _ROW + e * _N + s16 + srow
            idx_v[pl.ds(c * 16, 16)] = v
            return 0

        lax.fori_loop(0, ne // 16, idx_body, 0)
        for z in range(_AW // _ZW):
            pltpu.sync_copy(zeros_v, acc_shared.at[pl.ds(srow + z * _ZW, _ZW)])
        pltpu.sync_copy(ones_v, acc_shared.at[idx_v], add=True)
        pltpu.sync_copy(acc_shared.at[pl.ds(srow, _AW)], adj_hbm.at[g])
        return 0

    lax.fori_loop(0, _GPW, per_graph, 0)


def _build_adj_sc(sd):
    mesh = plsc.VectorSubcoreMesh(core_axis_name="c", subcore_axis_name="s",
                                  num_cores=_NC, num_subcores=_NS)
    kfn = pl.kernel(
        _adj_sc_body,
        out_type=jax.ShapeDtypeStruct((_B, _AW), jnp.float32),
        mesh=mesh,
        scratch_types=[
            pltpu.VMEM((2 * _ET * _E,), jnp.int32),
            pltpu.VMEM((_ET * _E,), jnp.int32),
            pltpu.VMEM((_ET * _E,), jnp.float32),
            pltpu.VMEM((_ZW,), jnp.float32),
            pltpu.VMEM_SHARED((_NS * _AW,), jnp.float32),
        ],
    )
    return kfn(sd).reshape(_B, _N, _ROW)


def _gnn_kernel(tab_ref, col_ref, adj_ref, We_ref, be_ref, Wi_ref, Wh_ref,
                bi_ref, bh_ref, out_ref, G):
    bf = jnp.bfloat16
    h = jnp.concatenate([tab_ref[...], col_ref[...]], axis=1)
    h = h.reshape(G * _N, _D)
    We = We_ref[...].astype(bf)
    be = be_ref[...]
    Wi = Wi_ref[...].astype(bf)
    Wh = Wh_ref[...].astype(bf)
    bi = bi_ref[...]
    bh = bh_ref[...]
    adj = adj_ref[...].astype(bf)
    for _ in range(_T):
        hb = h.astype(bf)
        ms = [jnp.dot(hb, We[e], preferred_element_type=jnp.float32)
              + be[e][None, :] for e in range(_ET)]
        a_rows = []
        for g in range(G):
            mcat = jnp.concatenate(
                [ms[e][g * _N:(g + 1) * _N, :] for e in range(_ET)],
                axis=0).astype(bf)
            a_rows.append(jnp.dot(adj[g], mcat,
                                  preferred_element_type=jnp.float32))
        a = jnp.concatenate(a_rows, axis=0)
        gi = jnp.dot(a.astype(bf), Wi, preferred_element_type=jnp.float32) + bi
        gh = jnp.dot(hb, Wh, preferred_element_type=jnp.float32) + bh
        r = jax.nn.sigmoid(gi[:, :_D] + gh[:, :_D])
        z = jax.nn.sigmoid(gi[:, _D:2 * _D] + gh[:, _D:2 * _D])
        n = jnp.tanh(gi[:, 2 * _D:] + r * gh[:, 2 * _D:])
        h = (1.0 - z) * n + z * h
    out_ref[...] = h.reshape(G, _N, _D)


def _run_gnn(tabemb, colemb, adj, We, be, Wi, Wh, bi, bh, G=8):
    nblk = _B // G
    return pl.pallas_call(
        functools.partial(_gnn_kernel, G=G),
        grid=(nblk,),
        in_specs=[
            pl.BlockSpec((G, _NT, _D), lambda i: (i, 0, 0)),
            pl.BlockSpec((G, _NCOL, _D), lambda i: (i, 0, 0)),
            pl.BlockSpec((G, _N, _ROW), lambda i: (i, 0, 0)),
            pl.BlockSpec((_ET, _D, _D), lambda i: (0, 0, 0)),
            pl.BlockSpec((_ET, _D), lambda i: (0, 0)),
            pl.BlockSpec((_D, 3 * _D), lambda i: (0, 0)),
            pl.BlockSpec((_D, 3 * _D), lambda i: (0, 0)),
            pl.BlockSpec((1, 3 * _D), lambda i: (0, 0)),
            pl.BlockSpec((1, 3 * _D), lambda i: (0, 0)),
        ],
        out_specs=pl.BlockSpec((G, _N, _D), lambda i: (i, 0, 0)),
        out_shape=jax.ShapeDtypeStruct((_B, _N, _D), jnp.float32),
    )(tabemb, colemb, adj, We, be, Wi, Wh, bi, bh)


def kernel(colemb, tabemb, graph_tensor, We, be, Wi, Wh, bi, bh):
    sd = jnp.moveaxis(graph_tensor.reshape(_B, _ET * _E, 2), 2, 1)
    adj = _build_adj_sc(sd.reshape(_B, 2 * _ET * _E))
    out = _run_gnn(tabemb, colemb, adj, We, be, Wi, Wh,
                   bi.reshape(1, -1), bh.reshape(1, -1))
    return out[:, _NT:, :], out[:, :_NT, :]
